# Initial kernel scaffold; baseline (speedup 1.0000x reference)
import functools

import jax
import jax.numpy as jnp
from jax import lax
from jax.experimental import pallas as pl
from jax.experimental.pallas import tpu as pltpu
from jax.experimental.pallas import tpu_sc as plsc

NC = 2
NS = 16
NW = NC * NS


def _chunk_size(per_w):
    for ch in range(128, 0, -8):
        if per_w % ch == 0:
            return ch
    raise ValueError(per_w)


@functools.lru_cache(maxsize=None)
def _make_seg_sum(n, d, e):
    per_w = e // NW
    ch = _chunk_size(per_w)
    nch = per_w // ch
    rows_per_tile = n // NS

    mesh = plsc.VectorSubcoreMesh(core_axis_name="c", subcore_axis_name="s")

    @functools.partial(
        pl.kernel,
        out_type=jax.ShapeDtypeStruct((NC, n, d), jnp.float32),
        mesh=mesh,
        scratch_types=[
            pltpu.VMEM((nch, ch), jnp.int32),
            pltpu.VMEM((nch, ch), jnp.int32),
            pltpu.VMEM((ch, d), jnp.float32),
            pltpu.VMEM_SHARED((n, d), jnp.float32),
            pltpu.SemaphoreType.DMA,
        ],
    )
    def seg_sum(x_hbm, src_hbm, dst_hbm, zeros_hbm, out_hbm,
                srcb, dstb, rows, agg, sem):
        c = lax.axis_index("c")
        s = lax.axis_index("s")
        wid = s * NC + c
        base_n = s * rows_per_tile

        pltpu.sync_copy(zeros_hbm.at[pl.ds(base_n, rows_per_tile)],
                        agg.at[pl.ds(base_n, rows_per_tile)])
        pltpu.sync_copy(src_hbm.at[wid], srcb)
        pltpu.sync_copy(dst_hbm.at[wid], dstb)
        plsc.subcore_barrier()

        @pl.loop(0, nch)
        def _(j):
            pltpu.async_copy(x_hbm.at[srcb.at[j]], rows, sem).wait()
            pltpu.sync_copy(rows, agg.at[dstb.at[j]], add=True)

        plsc.subcore_barrier()
        pltpu.sync_copy(agg.at[pl.ds(base_n, rows_per_tile)],
                        out_hbm.at[c].at[pl.ds(base_n, rows_per_tile)])

    return seg_sum, nch, ch


@functools.lru_cache(maxsize=None)
def _make_dense(n, d_in, d_out, relu):
    blk = 1000
    grid = (n // blk,)

    def body(a0_ref, a1_ref, x_ref, wr_ref, wo_ref, b_ref, o_ref):
        a = a0_ref[...] + a1_ref[...]
        acc = jnp.dot(a, wr_ref[...], preferred_element_type=jnp.float32)
        acc = acc + jnp.dot(x_ref[...], wo_ref[...],
                            preferred_element_type=jnp.float32)
        acc = acc + b_ref[...]
        if relu:
            acc = jnp.maximum(acc, 0.0)
        o_ref[...] = acc

    return pl.pallas_call(
        body,
        grid=grid,
        in_specs=[
            pl.BlockSpec((blk, d_in), lambda i: (i, 0)),
            pl.BlockSpec((blk, d_in), lambda i: (i, 0)),
            pl.BlockSpec((blk, d_in), lambda i: (i, 0)),
            pl.BlockSpec((d_in, d_out), lambda i: (0, 0)),
            pl.BlockSpec((d_in, d_out), lambda i: (0, 0)),
            pl.BlockSpec((1, d_out), lambda i: (0, 0)),
        ],
        out_specs=pl.BlockSpec((blk, d_out), lambda i: (i, 0)),
        out_shape=jax.ShapeDtypeStruct((n, d_out), jnp.float32),
    )


def kernel(x, edge_index, W1_rel, b1, W1_root, W2_rel, b2, W2_root):
    n, d = x.shape
    e = edge_index.shape[1]
    seg_sum, nch, ch = _make_seg_sum(n, d, e)
    src = edge_index[0].reshape(NW, nch, ch)
    dst = edge_index[1].reshape(NW, nch, ch)
    zeros = jnp.zeros((n, d), jnp.float32)

    p1 = seg_sum(x, src, dst, zeros)
    h = _make_dense(n, d, W1_rel.shape[0], True)(
        p1[0], p1[1], x, W1_rel.T, W1_root.T, b1[None, :])
    p2 = seg_sum(h, src, dst, zeros)
    out = _make_dense(n, d, W2_rel.shape[0], False)(
        p2[0], p2[1], h, W2_rel.T, W2_root.T, b2[None, :])
    return out


# speedup vs baseline: 7.1058x; 7.1058x over previous
import functools

import jax
import jax.numpy as jnp
from jax import lax
from jax.experimental import pallas as pl
from jax.experimental.pallas import tpu as pltpu
from jax.experimental.pallas import tpu_sc as plsc

NC = 2
NS = 16
NW = NC * NS


def _chunk_size(per_w):
    for ch in range(128, 0, -8):
        if per_w % ch == 0:
            return ch
    raise ValueError(per_w)


@functools.lru_cache(maxsize=None)
def _make_seg_sum(n, d, e):
    per_w = e // NW
    ch = _chunk_size(per_w)
    nch = per_w // ch
    rows_per_tile = -(-n // (NS * 8)) * 8
    n_pad = rows_per_tile * NS

    mesh = plsc.VectorSubcoreMesh(core_axis_name="c", subcore_axis_name="s")

    @functools.partial(
        pl.kernel,
        out_type=jax.ShapeDtypeStruct((NC, n_pad, d), jnp.float32),
        mesh=mesh,
        scratch_types=[
            pltpu.VMEM((nch, ch), jnp.int32),
            pltpu.VMEM((nch, ch), jnp.int32),
            pltpu.VMEM((ch, d), jnp.float32),
            pltpu.VMEM_SHARED((n_pad, d), jnp.float32),
            pltpu.SemaphoreType.DMA,
        ],
    )
    def seg_sum(x_hbm, src_hbm, dst_hbm, zeros_hbm, out_hbm,
                srcb, dstb, rows, agg, sem):
        c = lax.axis_index("c")
        s = lax.axis_index("s")
        wid = s * NC + c
        base_n = s * rows_per_tile

        pltpu.sync_copy(zeros_hbm.at[pl.ds(base_n, rows_per_tile)],
                        agg.at[pl.ds(base_n, rows_per_tile)])
        pltpu.sync_copy(src_hbm.at[wid], srcb)
        pltpu.sync_copy(dst_hbm.at[wid], dstb)
        plsc.subcore_barrier()

        @pl.loop(0, nch)
        def _(j):
            pltpu.async_copy(x_hbm.at[srcb.at[j]], rows, sem).wait()
            pltpu.sync_copy(rows, agg.at[dstb.at[j]], add=True)

        plsc.subcore_barrier()
        pltpu.sync_copy(agg.at[pl.ds(base_n, rows_per_tile)],
                        out_hbm.at[c].at[pl.ds(base_n, rows_per_tile)])

    return seg_sum, nch, ch, n_pad


@functools.lru_cache(maxsize=None)
def _make_dense(n, d_in, d_out, relu):
    blk = 1000
    grid = (n // blk,)

    def body(a0_ref, a1_ref, x_ref, wr_ref, wo_ref, b_ref, o_ref):
        a = a0_ref[...] + a1_ref[...]
        acc = jnp.dot(a, wr_ref[...], preferred_element_type=jnp.float32)
        acc = acc + jnp.dot(x_ref[...], wo_ref[...],
                            preferred_element_type=jnp.float32)
        acc = acc + b_ref[...]
        if relu:
            acc = jnp.maximum(acc, 0.0)
        o_ref[...] = acc

    return pl.pallas_call(
        body,
        grid=grid,
        in_specs=[
            pl.BlockSpec((blk, d_in), lambda i: (i, 0)),
            pl.BlockSpec((blk, d_in), lambda i: (i, 0)),
            pl.BlockSpec((blk, d_in), lambda i: (i, 0)),
            pl.BlockSpec((d_in, d_out), lambda i: (0, 0)),
            pl.BlockSpec((d_in, d_out), lambda i: (0, 0)),
            pl.BlockSpec((1, d_out), lambda i: (0, 0)),
        ],
        out_specs=pl.BlockSpec((blk, d_out), lambda i: (i, 0)),
        out_shape=jax.ShapeDtypeStruct((n, d_out), jnp.float32),
    )


def kernel(x, edge_index, W1_rel, b1, W1_root, W2_rel, b2, W2_root):
    n, d = x.shape
    e = edge_index.shape[1]
    seg_sum, nch, ch, n_pad = _make_seg_sum(n, d, e)
    src = edge_index[0].reshape(NW, nch, ch)
    dst = edge_index[1].reshape(NW, nch, ch)
    zeros = jnp.zeros((n_pad, d), jnp.float32)

    p1 = seg_sum(x, src, dst, zeros)
    h = _make_dense(n, d, W1_rel.shape[0], True)(
        p1[0], p1[1], x, W1_rel.T, W1_root.T, b1[None, :])
    p2 = seg_sum(h, src, dst, zeros)
    out = _make_dense(n, d, W2_rel.shape[0], False)(
        p2[0], p2[1], h, W2_rel.T, W2_root.T, b2[None, :])
    return out
